# baseline (device time: 66638 ns/iter reference)
import jax
import jax.numpy as jnp
from jax import lax
from jax.experimental import pallas as pl
from jax.experimental.pallas import tpu as pltpu

N_DEV = 32
M = 1024
N = 1024
CH = M // N_DEV


def kernel(A, B):
    def body(
        a_ref, b_ref, out_ref, p_ref, recv_ref,
        send_sems, recv_sems, ag_send_sems, ag_recv_sems,
    ):
        my = lax.axis_index("i")

        barrier = pltpu.get_barrier_semaphore()
        for d in range(1, N_DEV):
            pl.semaphore_signal(
                barrier, inc=1,
                device_id=((my + d) % N_DEV,),
                device_id_type=pl.DeviceIdType.MESH,
            )
        pl.semaphore_wait(barrier, N_DEV - 1)

        p_ref[...] = jnp.dot(
            a_ref[...].astype(jnp.bfloat16),
            b_ref[...].astype(jnp.bfloat16),
            preferred_element_type=jnp.float32,
        ).astype(jnp.bfloat16)

        p1_sends = []
        for d in range(1, N_DEV):
            t = (my + d) % N_DEV
            rdma = pltpu.make_async_remote_copy(
                src_ref=p_ref.at[pl.ds(t * CH, CH)],
                dst_ref=recv_ref.at[my],
                send_sem=send_sems.at[d - 1],
                recv_sem=recv_sems.at[my],
                device_id=(t,),
                device_id_type=pl.DeviceIdType.MESH,
            )
            rdma.start()
            p1_sends.append(rdma)

        recv_ref[pl.ds(my, 1)] = p_ref[pl.ds(my * CH, CH), :][None]

        for d in range(1, N_DEV):
            s = (my + N_DEV - d) % N_DEV
            rdma = pltpu.make_async_remote_copy(
                src_ref=recv_ref.at[s],
                dst_ref=recv_ref.at[s],
                send_sem=send_sems.at[d - 1],
                recv_sem=recv_sems.at[s],
                device_id=(s,),
                device_id_type=pl.DeviceIdType.MESH,
            )
            rdma.wait_recv()

        tot = jnp.sum(recv_ref[...].astype(jnp.float32), axis=0)
        g = jnp.maximum(tot, 0.0).astype(jnp.bfloat16)
        out_ref[pl.ds(my * CH, CH), :] = g

        p2_sends = []
        for d in range(1, N_DEV):
            t = (my + d) % N_DEV
            rdma = pltpu.make_async_remote_copy(
                src_ref=out_ref.at[pl.ds(my * CH, CH)],
                dst_ref=out_ref.at[pl.ds(my * CH, CH)],
                send_sem=ag_send_sems.at[d - 1],
                recv_sem=ag_recv_sems.at[my],
                device_id=(t,),
                device_id_type=pl.DeviceIdType.MESH,
            )
            rdma.start()
            p2_sends.append(rdma)

        for rdma in p1_sends:
            rdma.wait_send()

        for d in range(1, N_DEV):
            s = (my + N_DEV - d) % N_DEV
            rdma = pltpu.make_async_remote_copy(
                src_ref=out_ref.at[pl.ds(s * CH, CH)],
                dst_ref=out_ref.at[pl.ds(s * CH, CH)],
                send_sem=ag_send_sems.at[d - 1],
                recv_sem=ag_recv_sems.at[s],
                device_id=(s,),
                device_id_type=pl.DeviceIdType.MESH,
            )
            rdma.wait_recv()

        for rdma in p2_sends:
            rdma.wait_send()

    return pl.pallas_call(
        body,
        out_shape=jax.ShapeDtypeStruct((M, N), jnp.bfloat16),
        in_specs=[
            pl.BlockSpec(memory_space=pltpu.VMEM),
            pl.BlockSpec(memory_space=pltpu.VMEM),
        ],
        out_specs=pl.BlockSpec(memory_space=pltpu.VMEM),
        scratch_shapes=[
            pltpu.VMEM((M, N), jnp.bfloat16),
            pltpu.VMEM((N_DEV, CH, N), jnp.bfloat16),
            pltpu.SemaphoreType.DMA((N_DEV - 1,)),
            pltpu.SemaphoreType.DMA((N_DEV,)),
            pltpu.SemaphoreType.DMA((N_DEV - 1,)),
            pltpu.SemaphoreType.DMA((N_DEV,)),
        ],
        compiler_params=pltpu.CompilerParams(collective_id=0),
    )(A, B)


# device time: 63248 ns/iter; 1.0536x vs baseline; 1.0536x over previous
import jax
import jax.numpy as jnp
from jax import lax
from jax.experimental import pallas as pl
from jax.experimental.pallas import tpu as pltpu

N_DEV = 32
M = 1024
N = 1024
CH = M // N_DEV
NC = N // 2


def kernel(A, B):
    def body(
        a_ref, b_ref, out_ref, p_ref, recv0_ref, recv1_ref,
        s_sems0, s_sems1, r_sems0, r_sems1, a_sems0, a_sems1,
    ):
        my = lax.axis_index("i")

        barrier = pltpu.get_barrier_semaphore()
        for d in range(1, N_DEV):
            pl.semaphore_signal(
                barrier, inc=1,
                device_id=((my + d) % N_DEV,),
                device_id_type=pl.DeviceIdType.MESH,
            )
        pl.semaphore_wait(barrier, N_DEV - 1)

        p_ref[...] = jnp.dot(
            a_ref[...].astype(jnp.bfloat16),
            b_ref[...].astype(jnp.bfloat16),
            preferred_element_type=jnp.float32,
        ).astype(jnp.bfloat16)

        halves = (
            (0, recv0_ref, s_sems0, r_sems0, a_sems0),
            (1, recv1_ref, s_sems1, r_sems1, a_sems1),
        )

        rs_sends = {0: [], 1: []}
        for h, recv_ref, s_sems, r_sems, _ in halves:
            for d in range(1, N_DEV):
                t = (my + d) % N_DEV
                rdma = pltpu.make_async_remote_copy(
                    src_ref=p_ref.at[pl.ds(t * CH, CH), pl.ds(h * NC, NC)],
                    dst_ref=recv_ref.at[my],
                    send_sem=s_sems.at[d - 1],
                    recv_sem=r_sems.at[my],
                    device_id=(t,),
                    device_id_type=pl.DeviceIdType.MESH,
                )
                rdma.start()
                rs_sends[h].append(rdma)

        ag_sends = []
        for h, recv_ref, s_sems, r_sems, a_sems in halves:
            recv_ref[pl.ds(my, 1)] = p_ref[
                pl.ds(my * CH, CH), pl.ds(h * NC, NC)
            ][None]

            for d in range(1, N_DEV):
                s = (my + N_DEV - d) % N_DEV
                rdma = pltpu.make_async_remote_copy(
                    src_ref=recv_ref.at[s],
                    dst_ref=recv_ref.at[s],
                    send_sem=s_sems.at[d - 1],
                    recv_sem=r_sems.at[s],
                    device_id=(s,),
                    device_id_type=pl.DeviceIdType.MESH,
                )
                rdma.wait_recv()

            tot = jnp.sum(recv_ref[...].astype(jnp.float32), axis=0)
            g = jnp.maximum(tot, 0.0).astype(jnp.bfloat16)
            out_ref[pl.ds(my * CH, CH), pl.ds(h * NC, NC)] = g

            for rdma in rs_sends[h]:
                rdma.wait_send()
            for d in range(1, N_DEV):
                t = (my + d) % N_DEV
                rdma = pltpu.make_async_remote_copy(
                    src_ref=out_ref.at[pl.ds(my * CH, CH), pl.ds(h * NC, NC)],
                    dst_ref=out_ref.at[pl.ds(my * CH, CH), pl.ds(h * NC, NC)],
                    send_sem=s_sems.at[d - 1],
                    recv_sem=a_sems.at[my],
                    device_id=(t,),
                    device_id_type=pl.DeviceIdType.MESH,
                )
                rdma.start()
                ag_sends.append(rdma)

        for h, recv_ref, s_sems, r_sems, a_sems in halves:
            for d in range(1, N_DEV):
                s = (my + N_DEV - d) % N_DEV
                rdma = pltpu.make_async_remote_copy(
                    src_ref=out_ref.at[pl.ds(s * CH, CH), pl.ds(h * NC, NC)],
                    dst_ref=out_ref.at[pl.ds(s * CH, CH), pl.ds(h * NC, NC)],
                    send_sem=s_sems.at[d - 1],
                    recv_sem=a_sems.at[s],
                    device_id=(s,),
                    device_id_type=pl.DeviceIdType.MESH,
                )
                rdma.wait_recv()

        for rdma in ag_sends:
            rdma.wait_send()

    return pl.pallas_call(
        body,
        out_shape=jax.ShapeDtypeStruct((M, N), jnp.bfloat16),
        in_specs=[
            pl.BlockSpec(memory_space=pltpu.VMEM),
            pl.BlockSpec(memory_space=pltpu.VMEM),
        ],
        out_specs=pl.BlockSpec(memory_space=pltpu.VMEM),
        scratch_shapes=[
            pltpu.VMEM((M, N), jnp.bfloat16),
            pltpu.VMEM((N_DEV, CH, NC), jnp.bfloat16),
            pltpu.VMEM((N_DEV, CH, NC), jnp.bfloat16),
            pltpu.SemaphoreType.DMA((N_DEV - 1,)),
            pltpu.SemaphoreType.DMA((N_DEV - 1,)),
            pltpu.SemaphoreType.DMA((N_DEV,)),
            pltpu.SemaphoreType.DMA((N_DEV,)),
            pltpu.SemaphoreType.DMA((N_DEV,)),
            pltpu.SemaphoreType.DMA((N_DEV,)),
        ],
        compiler_params=pltpu.CompilerParams(collective_id=0),
    )(A, B)
